# Initial kernel scaffold; baseline (speedup 1.0000x reference)
import jax
import jax.numpy as jnp
from jax.experimental import pallas as pl


def kernel(x, pos, batch, edge_index_3rd, num_nodes_per_graph, W_in, b_in, Wg1, bg1, Wg2, bg2, Wmsg, Wupd, bupd, W_out, b_out, Wd1, bd1, Wd2, bd2, Wd3, bd3):
    raise NotImplementedError("write your pallas kernel here")


# speedup vs baseline: 1.7049x; 1.7049x over previous
import functools

import jax
import jax.numpy as jnp
from jax import lax
from jax.experimental import pallas as pl
from jax.experimental.pallas import tpu as pltpu
from jax.experimental.pallas import tpu_sc as plsc

F32 = jnp.float32


def _make_posgather(N, E, KG=None):
  EPT = E // 32
  KG = KG or next(k for k in (1000, 500, 250, 125) if EPT % k == 0)
  NBLK = EPT // KG
  mesh = plsc.VectorSubcoreMesh(core_axis_name="c", subcore_axis_name="s", num_cores=2, num_subcores=16)

  @functools.partial(
      pl.kernel,
      out_type=[jax.ShapeDtypeStruct((E, 4), F32) for _ in range(4)],
      mesh=mesh,
      compiler_params=pltpu.CompilerParams(use_tc_tiling_on_sc=False),
      scratch_types=[
          pltpu.VMEM((KG,), jnp.int32),
          pltpu.VMEM((KG, 16), F32),
          pltpu.SemaphoreType.DMA,
      ],
  )
  def posgather(pos16_hbm, ih0, ih1, ih2, ih3, o0, o1, o2, o3, idx_v, buf, sem):
    c = lax.axis_index("c")
    s = lax.axis_index("s")
    wid = s * 2 + c
    idxs = (ih0, ih1, ih2, ih3)
    outs = (o0, o1, o2, o3)

    def blk(b, carry):
      base = wid * EPT + b * KG
      for g in range(4):
        pltpu.sync_copy(idxs[g].at[pl.ds(base, KG)], idx_v)
        pltpu.async_copy(pos16_hbm.at[idx_v], buf, sem).wait()
        pltpu.sync_copy(buf.at[:, pl.ds(0, 4)], outs[g].at[pl.ds(base, KG)])
      return carry

    lax.fori_loop(0, NBLK, blk, 0)

  return posgather


def _make_msg(N, E, K=None):
  K = K or next(k for k in (200, 100, 50) if (E // 16) % k == 0)
  NBLK_E = (E // 16) // K
  NPT = N // 16
  mesh = plsc.VectorSubcoreMesh(core_axis_name="c", subcore_axis_name="s", num_cores=2, num_subcores=16)

  @functools.partial(
      pl.kernel,
      out_type=jax.ShapeDtypeStruct((2, N, 32), F32),
      mesh=mesh,
      compiler_params=pltpu.CompilerParams(use_tc_tiling_on_sc=False),
      scratch_types=[
          pltpu.VMEM((K,), jnp.int32),
          pltpu.VMEM((K,), jnp.int32),
          pltpu.VMEM((K, 32), F32),
          pltpu.VMEM((K, 32), F32),
          pltpu.VMEM((NPT // 25, 32), F32),
          pltpu.VMEM_SHARED((N, 32), F32),
          pltpu.SemaphoreType.DMA,
      ],
  )
  def msg(hw2, ghe2, jidx, iidx, out2, jv, iv, rows, ghb, zerob, agg_sp, sem):
    c = lax.axis_index("c")
    s = lax.axis_index("s")

    zch = NPT // 25

    def zset(r, carry):
      zero = jnp.zeros((16,), F32)
      for u in range(2):
        zerob[r, pl.ds(u * 16, 16)] = zero
      return carry

    lax.fori_loop(0, zch, zset, 0)
    for q in range(25):
      pltpu.sync_copy(zerob, agg_sp.at[pl.ds(s * NPT + q * zch, zch)])
    plsc.subcore_barrier()

    def edge_loop(hw_h, ghe_h):
      def blk(b, carry):
        base = s * (E // 16) + b * K
        pltpu.sync_copy(jidx.at[pl.ds(base, K)], jv)
        pltpu.sync_copy(iidx.at[pl.ds(base, K)], iv)
        pltpu.async_copy(hw_h.at[jv], rows, sem).wait()
        pltpu.sync_copy(ghe_h.at[pl.ds(base, K)], ghb)

        def mul(b8, carry2):
          for ee in range(8):
            e = b8 * 8 + ee
            for u in range(2):
              sl = pl.ds(u * 16, 16)
              rows[e, sl] = rows[e, sl] * ghb[e, sl]
          return carry2

        lax.fori_loop(0, K // 8, mul, 0)
        pltpu.sync_copy(rows, agg_sp.at[iv], add=True)
        return carry

      lax.fori_loop(0, NBLK_E, blk, 0)

    for cc in (0, 1):
      @pl.when(c == cc)
      def _():
        edge_loop(hw2.at[cc], ghe2.at[cc])

    plsc.subcore_barrier()

    for cc in (0, 1):
      @pl.when(c == cc)
      def _():
        pltpu.sync_copy(agg_sp.at[pl.ds(s * NPT, NPT)],
                        out2.at[cc, pl.ds(s * NPT, NPT)])

  return msg


def _enc_in_body(x_ref, win_ref, bin_ref, wmsg_ref, h_ref, hw_ref):
  h = jnp.dot(x_ref[...], win_ref[...], preferred_element_type=F32) + bin_ref[...]
  h_ref[...] = h
  hw = jnp.dot(h, wmsg_ref[...], preferred_element_type=F32)
  hw_ref[...] = jnp.stack([hw[:, 0:32], hw[:, 32:64]], axis=0)


def _geo_body(pi_ref, pj_ref, pk_ref, pl_ref, wg1t_ref, bg1_ref, wg2t_ref,
              bg2_ref, o0, o1, o2, *, cutoff):
  pi = pi_ref[...].T
  pj = pj_ref[...].T
  pk = pk_ref[...].T
  plr = pl_ref[...].T
  v1 = pi - pj
  v2 = pj - pk
  v3 = pk - plr
  eps = 1e-8

  def dot3(a, b):
    return jnp.sum(a[0:3, :] * b[0:3, :], axis=0, keepdims=True)

  def cross(a, b):
    ax, ay, az = a[0:1, :], a[1:2, :], a[2:3, :]
    bx, by, bz = b[0:1, :], b[1:2, :], b[2:3, :]
    return jnp.concatenate(
        [ay * bz - az * by, az * bx - ax * bz, ax * by - ay * bx], axis=0)

  d1 = jnp.sqrt(dot3(v1, v1) + eps)
  d2 = jnp.sqrt(dot3(v2, v2) + eps)
  d3 = jnp.sqrt(dot3(v3, v3) + eps)
  cos_a = dot3(v1, v2) / (d1 * d2)
  cos_b = dot3(v2, v3) / (d2 * d3)
  n1 = cross(v1, v2)
  n2 = cross(v2, v3)
  n1n = jnp.sqrt(dot3(n1, n1) + eps)
  n2n = jnp.sqrt(dot3(n2, n2) + eps)
  cos_t = dot3(n1, n2) / (n1n * n2n)
  geo = jnp.concatenate([d1, d2, d3, cos_a, cos_b, cos_t], axis=0)
  env = 0.5 * (jnp.cos(jnp.pi * jnp.minimum(d1, cutoff) / cutoff) + 1.0)
  env = env * (d1 < cutoff).astype(F32)

  for t, out in enumerate((o0, o1, o2)):
    g1 = jnp.maximum(
        jnp.dot(wg1t_ref[t], geo, preferred_element_type=F32)
        + bg1_ref[t][:, None], 0.0)
    g2 = (jnp.dot(wg2t_ref[t], g1, preferred_element_type=F32)
          + bg2_ref[t][:, None])
    ghe = g2 * env
    out[...] = jnp.stack([ghe[0:32, :].T, ghe[32:64, :].T], axis=0)


def _upd_body(h_ref, agg_ref, wupd_ref, bupd_ref, wmsg_ref, h_out, hw_out):
  agg = jnp.concatenate([agg_ref[0], agg_ref[1]], axis=1)
  upd = jnp.maximum(
      jnp.dot(agg, wupd_ref[...], preferred_element_type=F32) + bupd_ref[...],
      0.0)
  h = h_ref[...] + upd
  h_out[...] = h
  if hw_out is not None:
    hw = jnp.dot(h, wmsg_ref[...], preferred_element_type=F32)
    hw_out[...] = jnp.stack([hw[:, 0:32], hw[:, 32:64]], axis=0)


def _upd_body_last(h_ref, agg_ref, wupd_ref, bupd_ref, h_out):
  _upd_body(h_ref, agg_ref, wupd_ref, bupd_ref, None, h_out, None)


def _readout_body(h_ref, b_ref, wout_ref, bout_ref, z_ref, zg, *, nsteps, B):
  step = pl.program_id(0)
  onehot = (b_ref[...] == lax.broadcasted_iota(jnp.int32, (1, B), 1)
            ).astype(F32)
  contrib = lax.dot_general(onehot, h_ref[...], (((0,), (0,)), ((), ())),
                            preferred_element_type=F32)

  @pl.when(step == 0)
  def _():
    zg[...] = jnp.zeros_like(zg)

  zg[...] += contrib

  @pl.when(step == nsteps - 1)
  def _():
    z_ref[...] = (jnp.dot(zg[...], wout_ref[...], preferred_element_type=F32)
                  + bout_ref[...])


def _dec_body(z_ref, b_ref, wd1_ref, bd1_ref, wd2_ref, bd2_ref, wd3_ref,
              bd3_ref, out_ref, *, B):
  onehot = (b_ref[...] == lax.broadcasted_iota(jnp.int32, (1, B), 1)
            ).astype(F32)
  zx = jnp.dot(onehot, z_ref[...], preferred_element_type=F32)
  hd = jnp.maximum(
      jnp.dot(zx, wd1_ref[...], preferred_element_type=F32) + bd1_ref[...], 0.0)
  hd = jnp.maximum(
      jnp.dot(hd, wd2_ref[...], preferred_element_type=F32) + bd2_ref[...], 0.0)
  out_ref[...] = (jnp.dot(hd, wd3_ref[...], preferred_element_type=F32)
                  + bd3_ref[...])


def _full(shape, dtype=F32):
  return pl.BlockSpec(shape, lambda *_: tuple(0 for _ in shape))


def kernel(x, pos, batch, edge_index_3rd, num_nodes_per_graph,
           W_in, b_in, Wg1, bg1, Wg2, bg2, Wmsg, Wupd, bupd,
           W_out, b_out, Wd1, bd1, Wd2, bd2, Wd3, bd3):
  N, C_IN = x.shape
  E = edge_index_3rd.shape[1]
  H = W_in.shape[1]
  LATENT = W_out.shape[1]
  T = Wmsg.shape[0]
  B = num_nodes_per_graph.shape[0]
  CUTOFF = 10.0

  KN = next(k for k in (2000, 1600, 800, 400, 200, 100) if N % k == 0)
  NSTEPS = N // KN
  KT = next(k for k in (3200, 1600, 800, 400) if E % k == 0)
  ESTEPS = E // KT

  pos16 = jnp.pad(pos, ((0, 0), (0, 13)))
  batch_col = batch.reshape(N, 1)
  jidx = edge_index_3rd[1]
  iidx = edge_index_3rd[0]
  Wg1T = jnp.transpose(Wg1, (0, 2, 1))
  Wg2T = jnp.transpose(Wg2, (0, 2, 1))
  b_in_r = b_in.reshape(1, H)
  b_out_r = b_out.reshape(1, LATENT)
  bd1_r = bd1.reshape(1, -1)
  bd2_r = bd2.reshape(1, -1)
  bd3_r = bd3.reshape(1, -1)

  h0, hw0 = pl.pallas_call(
      _enc_in_body,
      grid=(NSTEPS,),
      in_specs=[
          pl.BlockSpec((KN, C_IN), lambda n: (n, 0)),
          _full((C_IN, H)),
          _full((1, H)),
          _full((H, H)),
      ],
      out_specs=[
          pl.BlockSpec((KN, H), lambda n: (n, 0)),
          pl.BlockSpec((2, KN, 32), lambda n: (0, n, 0)),
      ],
      out_shape=[
          jax.ShapeDtypeStruct((N, H), F32),
          jax.ShapeDtypeStruct((2, N, 32), F32),
      ],
  )(x, W_in, b_in_r, Wmsg[0])

  kidx = edge_index_3rd[2]
  lidx = edge_index_3rd[3]
  pgi, pgj, pgk, pgl = _make_posgather(N, E)(pos16, iidx, jidx, kidx, lidx)

  ghe_all = pl.pallas_call(
      functools.partial(_geo_body, cutoff=CUTOFF),
      grid=(ESTEPS,),
      in_specs=[
          pl.BlockSpec((KT, 4), lambda e: (e, 0)),
          pl.BlockSpec((KT, 4), lambda e: (e, 0)),
          pl.BlockSpec((KT, 4), lambda e: (e, 0)),
          pl.BlockSpec((KT, 4), lambda e: (e, 0)),
          _full((T, H, 6)),
          _full((T, H)),
          _full((T, H, H)),
          _full((T, H)),
      ],
      out_specs=[pl.BlockSpec((2, KT, 32), lambda e: (0, e, 0))] * 3,
      out_shape=[jax.ShapeDtypeStruct((2, E, 32), F32)] * 3,
  )(pgi, pgj, pgk, pgl, Wg1T, bg1, Wg2T, bg2)

  msg_call = _make_msg(N, E)
  h = h0
  hw2 = hw0
  for t in range(T):
    agg2 = msg_call(hw2, ghe_all[t], jidx, iidx)
    last = (t == T - 1)
    if not last:
      h, hw2 = pl.pallas_call(
          _upd_body,
          grid=(NSTEPS,),
          in_specs=[
              pl.BlockSpec((KN, H), lambda n: (n, 0)),
              pl.BlockSpec((2, KN, 32), lambda n: (0, n, 0)),
              _full((H, H)),
              _full((1, H)),
              _full((H, H)),
          ],
          out_specs=[
              pl.BlockSpec((KN, H), lambda n: (n, 0)),
              pl.BlockSpec((2, KN, 32), lambda n: (0, n, 0)),
          ],
          out_shape=[
              jax.ShapeDtypeStruct((N, H), F32),
              jax.ShapeDtypeStruct((2, N, 32), F32),
          ],
      )(h, agg2, Wupd[t], bupd[t].reshape(1, H), Wmsg[t + 1])
    else:
      h = pl.pallas_call(
          _upd_body_last,
          grid=(NSTEPS,),
          in_specs=[
              pl.BlockSpec((KN, H), lambda n: (n, 0)),
              pl.BlockSpec((2, KN, 32), lambda n: (0, n, 0)),
              _full((H, H)),
              _full((1, H)),
          ],
          out_specs=pl.BlockSpec((KN, H), lambda n: (n, 0)),
          out_shape=jax.ShapeDtypeStruct((N, H), F32),
      )(h, agg2, Wupd[t], bupd[t].reshape(1, H))

  z = pl.pallas_call(
      functools.partial(_readout_body, nsteps=NSTEPS, B=B),
      grid=(NSTEPS,),
      in_specs=[
          pl.BlockSpec((KN, H), lambda n: (n, 0)),
          pl.BlockSpec((KN, 1), lambda n: (n, 0)),
          _full((H, LATENT)),
          _full((1, LATENT)),
      ],
      out_specs=pl.BlockSpec((B, LATENT), lambda n: (0, 0)),
      out_shape=jax.ShapeDtypeStruct((B, LATENT), F32),
      scratch_shapes=[pltpu.VMEM((B, H), F32)],
  )(h, batch_col, W_out, b_out_r)

  x_recon = pl.pallas_call(
      functools.partial(_dec_body, B=B),
      grid=(NSTEPS,),
      in_specs=[
          _full((B, LATENT)),
          pl.BlockSpec((KN, 1), lambda n: (n, 0)),
          _full((LATENT, H)),
          _full((1, H)),
          _full((H, 2 * H)),
          _full((1, 2 * H)),
          _full((2 * H, C_IN)),
          _full((1, C_IN)),
      ],
      out_specs=pl.BlockSpec((KN, C_IN), lambda n: (n, 0)),
      out_shape=jax.ShapeDtypeStruct((N, C_IN), F32),
  )(z, batch_col, Wd1, bd1_r, Wd2, bd2_r, Wd3, bd3_r)

  return (x_recon, z)


# speedup vs baseline: 4.0372x; 2.3680x over previous
import functools

import jax
import jax.numpy as jnp
from jax import lax
from jax.experimental import pallas as pl
from jax.experimental.pallas import tpu as pltpu
from jax.experimental.pallas import tpu_sc as plsc

F32 = jnp.float32


def _make_posgather(N, E, KG=None):
  EPT = E // 32
  KG = KG or next(k for k in (1000, 500, 250, 125) if EPT % k == 0)
  NBLK = EPT // KG
  mesh = plsc.VectorSubcoreMesh(core_axis_name="c", subcore_axis_name="s", num_cores=2, num_subcores=16)

  @functools.partial(
      pl.kernel,
      out_type=[jax.ShapeDtypeStruct((E, 16), F32) for _ in range(4)],
      mesh=mesh,
      compiler_params=pltpu.CompilerParams(use_tc_tiling_on_sc=False),
      scratch_types=[
          [pltpu.VMEM((KG,), jnp.int32) for _ in range(4)],
          [pltpu.VMEM((KG, 16), F32) for _ in range(4)],
          pltpu.SemaphoreType.DMA,
          pltpu.SemaphoreType.DMA,
          pltpu.SemaphoreType.DMA,
      ],
  )
  def posgather(pos16_hbm, ih0, ih1, ih2, ih3, o0, o1, o2, o3, idx_v, bufs,
                sem_i, sem_g, sem_w):
    c = lax.axis_index("c")
    s = lax.axis_index("s")
    wid = s * 2 + c
    idxs = (ih0, ih1, ih2, ih3)
    outs = (o0, o1, o2, o3)

    def blk(b, carry):
      base = wid * EPT + b * KG
      ic = [pltpu.async_copy(idxs[g].at[pl.ds(base, KG)], idx_v[g], sem_i)
            for g in range(4)]
      for cp in ic:
        cp.wait()
      gc = [pltpu.async_copy(pos16_hbm.at[idx_v[g]], bufs[g], sem_g)
            for g in range(4)]
      for cp in gc:
        cp.wait()
      wc = [pltpu.async_copy(bufs[g], outs[g].at[pl.ds(base, KG)], sem_w)
            for g in range(4)]
      for cp in wc:
        cp.wait()
      return carry

    lax.fori_loop(0, NBLK, blk, 0)

  return posgather


def _make_msg(N, E, K=None):
  K = K or next(k for k in (400, 200, 100, 50) if (E // 16) % k == 0)
  NBLK_E = (E // 16) // K
  NPT = N // 16
  mesh = plsc.VectorSubcoreMesh(core_axis_name="c", subcore_axis_name="s", num_cores=2, num_subcores=16)

  @functools.partial(
      pl.kernel,
      out_type=jax.ShapeDtypeStruct((2, N, 32), F32),
      mesh=mesh,
      compiler_params=pltpu.CompilerParams(use_tc_tiling_on_sc=False),
      scratch_types=[
          pltpu.VMEM((K,), jnp.int32),
          pltpu.VMEM((K,), jnp.int32),
          pltpu.VMEM((K, 32), F32),
          pltpu.VMEM((K, 32), F32),
          pltpu.VMEM((NPT // 25, 32), F32),
          pltpu.VMEM_SHARED((N, 32), F32),
          pltpu.SemaphoreType.DMA,
      ],
  )
  def msg(hw2, ghe2, jidx, iidx, out2, jv, iv, rows, ghb, zerob, agg_sp, sem):
    c = lax.axis_index("c")
    s = lax.axis_index("s")

    zch = NPT // 25

    def zset(r, carry):
      zero = jnp.zeros((16,), F32)
      for u in range(2):
        zerob[r, pl.ds(u * 16, 16)] = zero
      return carry

    lax.fori_loop(0, zch, zset, 0)
    for q in range(25):
      pltpu.sync_copy(zerob, agg_sp.at[pl.ds(s * NPT + q * zch, zch)])
    plsc.subcore_barrier()

    def edge_loop(hw_h, ghe_h):
      def blk(b, carry):
        base = s * (E // 16) + b * K
        cj = pltpu.async_copy(jidx.at[pl.ds(base, K)], jv, sem)
        ci = pltpu.async_copy(iidx.at[pl.ds(base, K)], iv, sem)
        cg = pltpu.async_copy(ghe_h.at[pl.ds(base, K)], ghb, sem)
        cj.wait()
        cr = pltpu.async_copy(hw_h.at[jv], rows, sem)
        ci.wait()
        cg.wait()
        cr.wait()

        def mul(b8, carry2):
          for ee in range(8):
            e = b8 * 8 + ee
            for u in range(2):
              sl = pl.ds(u * 16, 16)
              rows[e, sl] = rows[e, sl] * ghb[e, sl]
          return carry2

        lax.fori_loop(0, K // 8, mul, 0)
        pltpu.sync_copy(rows, agg_sp.at[iv], add=True)
        return carry

      lax.fori_loop(0, NBLK_E, blk, 0)

    for cc in (0, 1):
      @pl.when(c == cc)
      def _():
        edge_loop(hw2.at[cc], ghe2.at[cc])

    plsc.subcore_barrier()

    for cc in (0, 1):
      @pl.when(c == cc)
      def _():
        pltpu.sync_copy(agg_sp.at[pl.ds(s * NPT, NPT)],
                        out2.at[cc, pl.ds(s * NPT, NPT)])

  return msg


def _enc_in_body(x_ref, win_ref, bin_ref, wmsg_ref, h_ref, hw_ref):
  h = jnp.dot(x_ref[...], win_ref[...], preferred_element_type=F32) + bin_ref[...]
  h_ref[...] = h
  hw = jnp.dot(h, wmsg_ref[...], preferred_element_type=F32)
  hw_ref[...] = jnp.stack([hw[:, 0:32], hw[:, 32:64]], axis=0)


def _geo_body(pi_ref, pj_ref, pk_ref, pl_ref, wg1t_ref, bg1_ref, wg2t_ref,
              bg2_ref, o0, o1, o2, *, cutoff):
  pi = pi_ref[...].T
  pj = pj_ref[...].T
  pk = pk_ref[...].T
  plr = pl_ref[...].T
  v1 = pi - pj
  v2 = pj - pk
  v3 = pk - plr
  eps = 1e-8

  def dot3(a, b):
    return jnp.sum(a[0:3, :] * b[0:3, :], axis=0, keepdims=True)

  def cross(a, b):
    ax, ay, az = a[0:1, :], a[1:2, :], a[2:3, :]
    bx, by, bz = b[0:1, :], b[1:2, :], b[2:3, :]
    return jnp.concatenate(
        [ay * bz - az * by, az * bx - ax * bz, ax * by - ay * bx], axis=0)

  d1 = jnp.sqrt(dot3(v1, v1) + eps)
  d2 = jnp.sqrt(dot3(v2, v2) + eps)
  d3 = jnp.sqrt(dot3(v3, v3) + eps)
  cos_a = dot3(v1, v2) / (d1 * d2)
  cos_b = dot3(v2, v3) / (d2 * d3)
  n1 = cross(v1, v2)
  n2 = cross(v2, v3)
  n1n = jnp.sqrt(dot3(n1, n1) + eps)
  n2n = jnp.sqrt(dot3(n2, n2) + eps)
  cos_t = dot3(n1, n2) / (n1n * n2n)
  geo = jnp.concatenate([d1, d2, d3, cos_a, cos_b, cos_t], axis=0)
  env = 0.5 * (jnp.cos(jnp.pi * jnp.minimum(d1, cutoff) / cutoff) + 1.0)
  env = env * (d1 < cutoff).astype(F32)

  for t, out in enumerate((o0, o1, o2)):
    g1 = jnp.maximum(
        jnp.dot(wg1t_ref[t], geo, preferred_element_type=F32)
        + bg1_ref[t][:, None], 0.0)
    g2 = (jnp.dot(wg2t_ref[t], g1, preferred_element_type=F32)
          + bg2_ref[t][:, None])
    ghe = g2 * env
    out[...] = jnp.stack([ghe[0:32, :].T, ghe[32:64, :].T], axis=0)


def _upd_body(h_ref, agg_ref, wupd_ref, bupd_ref, wmsg_ref, h_out, hw_out):
  agg = jnp.concatenate([agg_ref[0], agg_ref[1]], axis=1)
  upd = jnp.maximum(
      jnp.dot(agg, wupd_ref[...], preferred_element_type=F32) + bupd_ref[...],
      0.0)
  h = h_ref[...] + upd
  h_out[...] = h
  if hw_out is not None:
    hw = jnp.dot(h, wmsg_ref[...], preferred_element_type=F32)
    hw_out[...] = jnp.stack([hw[:, 0:32], hw[:, 32:64]], axis=0)


def _upd_body_last(h_ref, agg_ref, wupd_ref, bupd_ref, h_out):
  _upd_body(h_ref, agg_ref, wupd_ref, bupd_ref, None, h_out, None)


def _readout_body(h_ref, b_ref, wout_ref, bout_ref, z_ref, zg, *, nsteps, B):
  step = pl.program_id(0)
  onehot = (b_ref[...] == lax.broadcasted_iota(jnp.int32, (1, B), 1)
            ).astype(F32)
  contrib = lax.dot_general(onehot, h_ref[...], (((0,), (0,)), ((), ())),
                            preferred_element_type=F32)

  @pl.when(step == 0)
  def _():
    zg[...] = jnp.zeros_like(zg)

  zg[...] += contrib

  @pl.when(step == nsteps - 1)
  def _():
    z_ref[...] = (jnp.dot(zg[...], wout_ref[...], preferred_element_type=F32)
                  + bout_ref[...])


def _dec_body(z_ref, b_ref, wd1_ref, bd1_ref, wd2_ref, bd2_ref, wd3_ref,
              bd3_ref, out_ref, *, B):
  onehot = (b_ref[...] == lax.broadcasted_iota(jnp.int32, (1, B), 1)
            ).astype(F32)
  zx = jnp.dot(onehot, z_ref[...], preferred_element_type=F32)
  hd = jnp.maximum(
      jnp.dot(zx, wd1_ref[...], preferred_element_type=F32) + bd1_ref[...], 0.0)
  hd = jnp.maximum(
      jnp.dot(hd, wd2_ref[...], preferred_element_type=F32) + bd2_ref[...], 0.0)
  out_ref[...] = (jnp.dot(hd, wd3_ref[...], preferred_element_type=F32)
                  + bd3_ref[...])


def _full(shape, dtype=F32):
  return pl.BlockSpec(shape, lambda *_: tuple(0 for _ in shape))


def kernel(x, pos, batch, edge_index_3rd, num_nodes_per_graph,
           W_in, b_in, Wg1, bg1, Wg2, bg2, Wmsg, Wupd, bupd,
           W_out, b_out, Wd1, bd1, Wd2, bd2, Wd3, bd3):
  N, C_IN = x.shape
  E = edge_index_3rd.shape[1]
  H = W_in.shape[1]
  LATENT = W_out.shape[1]
  T = Wmsg.shape[0]
  B = num_nodes_per_graph.shape[0]
  CUTOFF = 10.0

  KN = next(k for k in (2000, 1600, 800, 400, 200, 100) if N % k == 0)
  NSTEPS = N // KN
  KT = next(k for k in (3200, 1600, 800, 400) if E % k == 0)
  ESTEPS = E // KT

  pos16 = jnp.pad(pos, ((0, 0), (0, 13)))
  batch_col = batch.reshape(N, 1)
  jidx = edge_index_3rd[1]
  iidx = edge_index_3rd[0]
  Wg1T = jnp.transpose(Wg1, (0, 2, 1))
  Wg2T = jnp.transpose(Wg2, (0, 2, 1))
  b_in_r = b_in.reshape(1, H)
  b_out_r = b_out.reshape(1, LATENT)
  bd1_r = bd1.reshape(1, -1)
  bd2_r = bd2.reshape(1, -1)
  bd3_r = bd3.reshape(1, -1)

  h0, hw0 = pl.pallas_call(
      _enc_in_body,
      grid=(NSTEPS,),
      in_specs=[
          pl.BlockSpec((KN, C_IN), lambda n: (n, 0)),
          _full((C_IN, H)),
          _full((1, H)),
          _full((H, H)),
      ],
      out_specs=[
          pl.BlockSpec((KN, H), lambda n: (n, 0)),
          pl.BlockSpec((2, KN, 32), lambda n: (0, n, 0)),
      ],
      out_shape=[
          jax.ShapeDtypeStruct((N, H), F32),
          jax.ShapeDtypeStruct((2, N, 32), F32),
      ],
  )(x, W_in, b_in_r, Wmsg[0])

  kidx = edge_index_3rd[2]
  lidx = edge_index_3rd[3]
  pgi, pgj, pgk, pgl = _make_posgather(N, E)(pos16, iidx, jidx, kidx, lidx)

  ghe_all = pl.pallas_call(
      functools.partial(_geo_body, cutoff=CUTOFF),
      grid=(ESTEPS,),
      in_specs=[
          pl.BlockSpec((KT, 16), lambda e: (e, 0)),
          pl.BlockSpec((KT, 16), lambda e: (e, 0)),
          pl.BlockSpec((KT, 16), lambda e: (e, 0)),
          pl.BlockSpec((KT, 16), lambda e: (e, 0)),
          _full((T, H, 6)),
          _full((T, H)),
          _full((T, H, H)),
          _full((T, H)),
      ],
      out_specs=[pl.BlockSpec((2, KT, 32), lambda e: (0, e, 0))] * 3,
      out_shape=[jax.ShapeDtypeStruct((2, E, 32), F32)] * 3,
  )(pgi, pgj, pgk, pgl, Wg1T, bg1, Wg2T, bg2)

  msg_call = _make_msg(N, E)
  h = h0
  hw2 = hw0
  for t in range(T):
    agg2 = msg_call(hw2, ghe_all[t], jidx, iidx)
    last = (t == T - 1)
    if not last:
      h, hw2 = pl.pallas_call(
          _upd_body,
          grid=(NSTEPS,),
          in_specs=[
              pl.BlockSpec((KN, H), lambda n: (n, 0)),
              pl.BlockSpec((2, KN, 32), lambda n: (0, n, 0)),
              _full((H, H)),
              _full((1, H)),
              _full((H, H)),
          ],
          out_specs=[
              pl.BlockSpec((KN, H), lambda n: (n, 0)),
              pl.BlockSpec((2, KN, 32), lambda n: (0, n, 0)),
          ],
          out_shape=[
              jax.ShapeDtypeStruct((N, H), F32),
              jax.ShapeDtypeStruct((2, N, 32), F32),
          ],
      )(h, agg2, Wupd[t], bupd[t].reshape(1, H), Wmsg[t + 1])
    else:
      h = pl.pallas_call(
          _upd_body_last,
          grid=(NSTEPS,),
          in_specs=[
              pl.BlockSpec((KN, H), lambda n: (n, 0)),
              pl.BlockSpec((2, KN, 32), lambda n: (0, n, 0)),
              _full((H, H)),
              _full((1, H)),
          ],
          out_specs=pl.BlockSpec((KN, H), lambda n: (n, 0)),
          out_shape=jax.ShapeDtypeStruct((N, H), F32),
      )(h, agg2, Wupd[t], bupd[t].reshape(1, H))

  z = pl.pallas_call(
      functools.partial(_readout_body, nsteps=NSTEPS, B=B),
      grid=(NSTEPS,),
      in_specs=[
          pl.BlockSpec((KN, H), lambda n: (n, 0)),
          pl.BlockSpec((KN, 1), lambda n: (n, 0)),
          _full((H, LATENT)),
          _full((1, LATENT)),
      ],
      out_specs=pl.BlockSpec((B, LATENT), lambda n: (0, 0)),
      out_shape=jax.ShapeDtypeStruct((B, LATENT), F32),
      scratch_shapes=[pltpu.VMEM((B, H), F32)],
  )(h, batch_col, W_out, b_out_r)

  x_recon = pl.pallas_call(
      functools.partial(_dec_body, B=B),
      grid=(NSTEPS,),
      in_specs=[
          _full((B, LATENT)),
          pl.BlockSpec((KN, 1), lambda n: (n, 0)),
          _full((LATENT, H)),
          _full((1, H)),
          _full((H, 2 * H)),
          _full((1, 2 * H)),
          _full((2 * H, C_IN)),
          _full((1, C_IN)),
      ],
      out_specs=pl.BlockSpec((KN, C_IN), lambda n: (n, 0)),
      out_shape=jax.ShapeDtypeStruct((N, C_IN), F32),
  )(z, batch_col, Wd1, bd1_r, Wd2, bd2_r, Wd3, bd3_r)

  return (x_recon, z)


# speedup vs baseline: 4.0597x; 1.0056x over previous
import functools

import jax
import jax.numpy as jnp
from jax import lax
from jax.experimental import pallas as pl
from jax.experimental.pallas import tpu as pltpu
from jax.experimental.pallas import tpu_sc as plsc

F32 = jnp.float32


def _make_posgather(N, E, KG=None):
  EPT = E // 32
  KG = KG or next(k for k in (1000, 500, 250, 125) if EPT % k == 0)
  NBLK = EPT // KG
  mesh = plsc.VectorSubcoreMesh(core_axis_name="c", subcore_axis_name="s", num_cores=2, num_subcores=16)

  @functools.partial(
      pl.kernel,
      out_type=[jax.ShapeDtypeStruct((E, 16), F32) for _ in range(4)],
      mesh=mesh,
      compiler_params=pltpu.CompilerParams(use_tc_tiling_on_sc=False),
      scratch_types=[
          [pltpu.VMEM((KG,), jnp.int32) for _ in range(4)],
          [pltpu.VMEM((KG, 16), F32) for _ in range(4)],
          pltpu.SemaphoreType.DMA,
          pltpu.SemaphoreType.DMA,
          pltpu.SemaphoreType.DMA,
      ],
  )
  def posgather(pos16_hbm, ih0, ih1, ih2, ih3, o0, o1, o2, o3, idx_v, bufs,
                sem_i, sem_g, sem_w):
    c = lax.axis_index("c")
    s = lax.axis_index("s")
    wid = s * 2 + c
    idxs = (ih0, ih1, ih2, ih3)
    outs = (o0, o1, o2, o3)

    def blk(b, carry):
      base = wid * EPT + b * KG
      ic = [pltpu.async_copy(idxs[g].at[pl.ds(base, KG)], idx_v[g], sem_i)
            for g in range(4)]
      for cp in ic:
        cp.wait()
      gc = [pltpu.async_copy(pos16_hbm.at[idx_v[g]], bufs[g], sem_g)
            for g in range(4)]
      for cp in gc:
        cp.wait()
      wc = [pltpu.async_copy(bufs[g], outs[g].at[pl.ds(base, KG)], sem_w)
            for g in range(4)]
      for cp in wc:
        cp.wait()
      return carry

    lax.fori_loop(0, NBLK, blk, 0)

  return posgather


def _make_msg(N, E, K=None):
  K = K or next(k for k in (400, 200, 100, 50) if (E // 16) % k == 0)
  NBLK_E = (E // 16) // K
  NPT = N // 16
  mesh = plsc.VectorSubcoreMesh(core_axis_name="c", subcore_axis_name="s", num_cores=2, num_subcores=16)

  @functools.partial(
      pl.kernel,
      out_type=jax.ShapeDtypeStruct((2, N, 32), F32),
      mesh=mesh,
      compiler_params=pltpu.CompilerParams(use_tc_tiling_on_sc=False),
      scratch_types=[
          pltpu.VMEM((K,), jnp.int32),
          pltpu.VMEM((K,), jnp.int32),
          pltpu.VMEM((K, 32), F32),
          pltpu.VMEM((K, 32), F32),
          pltpu.VMEM((NPT // 25, 32), F32),
          pltpu.VMEM_SHARED((N, 32), F32),
          pltpu.SemaphoreType.DMA,
      ],
  )
  def msg(hw2, ghe2, jidx, iidx, out2, jv, iv, rows, ghb, zerob, agg_sp, sem):
    c = lax.axis_index("c")
    s = lax.axis_index("s")

    zch = NPT // 25

    def zset(r, carry):
      zero = jnp.zeros((16,), F32)
      for u in range(2):
        zerob[r, pl.ds(u * 16, 16)] = zero
      return carry

    lax.fori_loop(0, zch, zset, 0)
    for q in range(25):
      pltpu.sync_copy(zerob, agg_sp.at[pl.ds(s * NPT + q * zch, zch)])
    plsc.subcore_barrier()

    def edge_loop(hw_h, ghe_h):
      def blk(b, carry):
        base = s * (E // 16) + b * K
        cj = pltpu.async_copy(jidx.at[pl.ds(base, K)], jv, sem)
        ci = pltpu.async_copy(iidx.at[pl.ds(base, K)], iv, sem)
        cg = pltpu.async_copy(ghe_h.at[pl.ds(base, K)], ghb, sem)
        cj.wait()
        cr = pltpu.async_copy(hw_h.at[jv], rows, sem)
        ci.wait()
        cg.wait()
        cr.wait()

        def mul(b8, carry2):
          for ee in range(8):
            e = b8 * 8 + ee
            for u in range(2):
              sl = pl.ds(u * 16, 16)
              rows[e, sl] = rows[e, sl] * ghb[e, sl]
          return carry2

        lax.fori_loop(0, K // 8, mul, 0)
        pltpu.sync_copy(rows, agg_sp.at[iv], add=True)
        return carry

      lax.fori_loop(0, NBLK_E, blk, 0)

    for cc in (0, 1):
      @pl.when(c == cc)
      def _():
        edge_loop(hw2.at[cc], ghe2.at[cc])

    plsc.subcore_barrier()

    for cc in (0, 1):
      @pl.when(c == cc)
      def _():
        pltpu.sync_copy(agg_sp.at[pl.ds(s * NPT, NPT)],
                        out2.at[cc, pl.ds(s * NPT, NPT)])

  return msg


def _enc_in_body(x_ref, win_ref, bin_ref, wmsg_ref, h_ref, hw_ref):
  h = jnp.dot(x_ref[...], win_ref[...], preferred_element_type=F32) + bin_ref[...]
  h_ref[...] = h
  hw = jnp.dot(h, wmsg_ref[...], preferred_element_type=F32)
  hw_ref[...] = jnp.stack([hw[:, 0:32], hw[:, 32:64]], axis=0)


def _geo_body(pi_ref, pj_ref, pk_ref, pl_ref, w1_ref, b1_ref, w2_ref,
              b2_ref, senv_ref, o0, o1, o2, *, cutoff):
  pi = pi_ref[...].T
  pj = pj_ref[...].T
  pk = pk_ref[...].T
  plr = pl_ref[...].T
  v1 = pi - pj
  v2 = pj - pk
  v3 = pk - plr
  eps = 1e-8

  def dot3(a, b):
    return jnp.sum(a[0:3, :] * b[0:3, :], axis=0, keepdims=True)

  def cross(a, b):
    ax, ay, az = a[0:1, :], a[1:2, :], a[2:3, :]
    bx, by, bz = b[0:1, :], b[1:2, :], b[2:3, :]
    return jnp.concatenate(
        [ay * bz - az * by, az * bx - ax * bz, ax * by - ay * bx], axis=0)

  d1 = jnp.sqrt(dot3(v1, v1) + eps)
  d2 = jnp.sqrt(dot3(v2, v2) + eps)
  d3 = jnp.sqrt(dot3(v3, v3) + eps)
  cos_a = dot3(v1, v2) / (d1 * d2)
  cos_b = dot3(v2, v3) / (d2 * d3)
  n1 = cross(v1, v2)
  n2 = cross(v2, v3)
  n1n = jnp.sqrt(dot3(n1, n1) + eps)
  n2n = jnp.sqrt(dot3(n2, n2) + eps)
  cos_t = dot3(n1, n2) / (n1n * n2n)
  env = 0.5 * (jnp.cos(jnp.pi * jnp.minimum(d1, cutoff) / cutoff) + 1.0)
  env = env * (d1 < cutoff).astype(F32)
  zero = jnp.zeros_like(d1)
  g8 = jnp.concatenate([d1, d2, d3, cos_a, cos_b, cos_t, env, zero],
                       axis=0)
  kh = g8.shape[1] // 2
  geo_p = jnp.concatenate([g8[:, 0:kh], g8[:, kh:]], axis=0).T
  env_p = jnp.dot(geo_p, senv_ref[...], preferred_element_type=F32)
  for t, out in enumerate((o0, o1, o2)):
    g1 = jnp.maximum(
        jnp.dot(geo_p, w1_ref[t], preferred_element_type=F32)
        + b1_ref[t][None, :], 0.0)
    g2 = (jnp.dot(g1, w2_ref[t], preferred_element_type=F32)
          + b2_ref[t][None, :])
    ghe = g2 * env_p
    out[...] = jnp.stack([
        jnp.concatenate([ghe[:, 0:32], ghe[:, 64:96]], axis=0),
        jnp.concatenate([ghe[:, 32:64], ghe[:, 96:128]], axis=0)], axis=0)


def _upd_body(h_ref, agg_ref, wupd_ref, bupd_ref, wmsg_ref, h_out, hw_out):
  agg = jnp.concatenate([agg_ref[0], agg_ref[1]], axis=1)
  upd = jnp.maximum(
      jnp.dot(agg, wupd_ref[...], preferred_element_type=F32) + bupd_ref[...],
      0.0)
  h = h_ref[...] + upd
  h_out[...] = h
  if hw_out is not None:
    hw = jnp.dot(h, wmsg_ref[...], preferred_element_type=F32)
    hw_out[...] = jnp.stack([hw[:, 0:32], hw[:, 32:64]], axis=0)


def _tail_body(h_ref, agg_ref, b_ref, wupd_ref, bupd_ref, wout_ref, bout_ref,
               wd1_ref, bd1_ref, wd2_ref, bd2_ref, wd3_ref, bd3_ref,
               z_ref, xr_ref, zg, *, nsteps, B):
  ph = pl.program_id(0)
  n = pl.program_id(1)
  onehot = (b_ref[...] == lax.broadcasted_iota(jnp.int32, (1, B), 1)
            ).astype(F32)

  @pl.when(ph == 0)
  def _():
    agg = jnp.concatenate([agg_ref[0], agg_ref[1]], axis=1)
    upd = jnp.maximum(
        jnp.dot(agg, wupd_ref[...], preferred_element_type=F32)
        + bupd_ref[...], 0.0)
    h = h_ref[...] + upd
    contrib = lax.dot_general(onehot, h, (((0,), (0,)), ((), ())),
                              preferred_element_type=F32)

    @pl.when(n == 0)
    def _():
      zg[...] = jnp.zeros_like(zg)

    zg[...] += contrib

    @pl.when(n == nsteps - 1)
    def _():
      z_ref[...] = (jnp.dot(zg[...], wout_ref[...],
                            preferred_element_type=F32) + bout_ref[...])

  @pl.when(ph == 1)
  def _():
    z = (jnp.dot(zg[...], wout_ref[...], preferred_element_type=F32)
         + bout_ref[...])
    zx = jnp.dot(onehot, z, preferred_element_type=F32)
    hd = jnp.maximum(
        jnp.dot(zx, wd1_ref[...], preferred_element_type=F32)
        + bd1_ref[...], 0.0)
    hd = jnp.maximum(
        jnp.dot(hd, wd2_ref[...], preferred_element_type=F32)
        + bd2_ref[...], 0.0)
    xr_ref[...] = (jnp.dot(hd, wd3_ref[...], preferred_element_type=F32)
                   + bd3_ref[...])


def _readout_body(h_ref, b_ref, wout_ref, bout_ref, z_ref, zg, *, nsteps, B):
  step = pl.program_id(0)
  onehot = (b_ref[...] == lax.broadcasted_iota(jnp.int32, (1, B), 1)
            ).astype(F32)
  contrib = lax.dot_general(onehot, h_ref[...], (((0,), (0,)), ((), ())),
                            preferred_element_type=F32)

  @pl.when(step == 0)
  def _():
    zg[...] = jnp.zeros_like(zg)

  zg[...] += contrib

  @pl.when(step == nsteps - 1)
  def _():
    z_ref[...] = (jnp.dot(zg[...], wout_ref[...], preferred_element_type=F32)
                  + bout_ref[...])


def _dec_body(z_ref, b_ref, wd1_ref, bd1_ref, wd2_ref, bd2_ref, wd3_ref,
              bd3_ref, out_ref, *, B):
  onehot = (b_ref[...] == lax.broadcasted_iota(jnp.int32, (1, B), 1)
            ).astype(F32)
  zx = jnp.dot(onehot, z_ref[...], preferred_element_type=F32)
  hd = jnp.maximum(
      jnp.dot(zx, wd1_ref[...], preferred_element_type=F32) + bd1_ref[...], 0.0)
  hd = jnp.maximum(
      jnp.dot(hd, wd2_ref[...], preferred_element_type=F32) + bd2_ref[...], 0.0)
  out_ref[...] = (jnp.dot(hd, wd3_ref[...], preferred_element_type=F32)
                  + bd3_ref[...])


def _full(shape, dtype=F32):
  return pl.BlockSpec(shape, lambda *_: tuple(0 for _ in shape))


def kernel(x, pos, batch, edge_index_3rd, num_nodes_per_graph,
           W_in, b_in, Wg1, bg1, Wg2, bg2, Wmsg, Wupd, bupd,
           W_out, b_out, Wd1, bd1, Wd2, bd2, Wd3, bd3):
  N, C_IN = x.shape
  E = edge_index_3rd.shape[1]
  H = W_in.shape[1]
  LATENT = W_out.shape[1]
  T = Wmsg.shape[0]
  B = num_nodes_per_graph.shape[0]
  CUTOFF = 10.0

  KN = next(k for k in (2000, 1600, 800, 400, 200, 100) if N % k == 0)
  NSTEPS = N // KN
  KT = next(k for k in (3200, 1600, 800, 400) if E % k == 0)
  ESTEPS = E // KT

  pos16 = jnp.pad(pos, ((0, 0), (0, 13)))
  batch_col = batch.reshape(N, 1)
  jidx = edge_index_3rd[1]
  iidx = edge_index_3rd[0]
  def blockdiag2(w):
    z = jnp.zeros_like(w)
    return jnp.concatenate([
        jnp.concatenate([w, z], axis=1),
        jnp.concatenate([z, w], axis=1)], axis=0)

  Wg1p = jnp.pad(Wg1, ((0, 0), (0, 2), (0, 0)))
  W1big = jnp.stack([blockdiag2(Wg1p[t]) for t in range(T)])
  W2big = jnp.stack([blockdiag2(Wg2[t]) for t in range(T)])
  b1big = jnp.concatenate([bg1, bg1], axis=1)
  b2big = jnp.concatenate([bg2, bg2], axis=1)
  Senv = jnp.zeros((16, 2 * H), F32)
  Senv = Senv.at[6, 0:H].set(1.0).at[14, H:2 * H].set(1.0)
  b_in_r = b_in.reshape(1, H)
  b_out_r = b_out.reshape(1, LATENT)
  bd1_r = bd1.reshape(1, -1)
  bd2_r = bd2.reshape(1, -1)
  bd3_r = bd3.reshape(1, -1)

  h0, hw0 = pl.pallas_call(
      _enc_in_body,
      grid=(NSTEPS,),
      in_specs=[
          pl.BlockSpec((KN, C_IN), lambda n: (n, 0)),
          _full((C_IN, H)),
          _full((1, H)),
          _full((H, H)),
      ],
      out_specs=[
          pl.BlockSpec((KN, H), lambda n: (n, 0)),
          pl.BlockSpec((2, KN, 32), lambda n: (0, n, 0)),
      ],
      out_shape=[
          jax.ShapeDtypeStruct((N, H), F32),
          jax.ShapeDtypeStruct((2, N, 32), F32),
      ],
  )(x, W_in, b_in_r, Wmsg[0])

  kidx = edge_index_3rd[2]
  lidx = edge_index_3rd[3]
  pgi, pgj, pgk, pgl = _make_posgather(N, E)(pos16, iidx, jidx, kidx, lidx)

  ghe_all = pl.pallas_call(
      functools.partial(_geo_body, cutoff=CUTOFF),
      grid=(ESTEPS,),
      in_specs=[
          pl.BlockSpec((KT, 16), lambda e: (e, 0)),
          pl.BlockSpec((KT, 16), lambda e: (e, 0)),
          pl.BlockSpec((KT, 16), lambda e: (e, 0)),
          pl.BlockSpec((KT, 16), lambda e: (e, 0)),
          _full((T, 16, 2 * H)),
          _full((T, 2 * H)),
          _full((T, 2 * H, 2 * H)),
          _full((T, 2 * H)),
          _full((16, 2 * H)),
      ],
      out_specs=[pl.BlockSpec((2, KT, 32), lambda e: (0, e, 0))] * 3,
      out_shape=[jax.ShapeDtypeStruct((2, E, 32), F32)] * 3,
  )(pgi, pgj, pgk, pgl, W1big, b1big, W2big, b2big, Senv)

  msg_call = _make_msg(N, E)
  h = h0
  hw2 = hw0
  for t in range(T):
    agg2 = msg_call(hw2, ghe_all[t], jidx, iidx)
    last = (t == T - 1)
    if not last:
      h, hw2 = pl.pallas_call(
          _upd_body,
          grid=(NSTEPS,),
          in_specs=[
              pl.BlockSpec((KN, H), lambda n: (n, 0)),
              pl.BlockSpec((2, KN, 32), lambda n: (0, n, 0)),
              _full((H, H)),
              _full((1, H)),
              _full((H, H)),
          ],
          out_specs=[
              pl.BlockSpec((KN, H), lambda n: (n, 0)),
              pl.BlockSpec((2, KN, 32), lambda n: (0, n, 0)),
          ],
          out_shape=[
              jax.ShapeDtypeStruct((N, H), F32),
              jax.ShapeDtypeStruct((2, N, 32), F32),
          ],
      )(h, agg2, Wupd[t], bupd[t].reshape(1, H), Wmsg[t + 1])
    else:
      z, x_recon = pl.pallas_call(
          functools.partial(_tail_body, nsteps=NSTEPS, B=B),
          grid=(2, NSTEPS),
          in_specs=[
              pl.BlockSpec((KN, H), lambda p, n: (n, 0)),
              pl.BlockSpec((2, KN, 32), lambda p, n: (0, n, 0)),
              pl.BlockSpec((KN, 1), lambda p, n: (n, 0)),
              _full((H, H)),
              _full((1, H)),
              _full((H, LATENT)),
              _full((1, LATENT)),
              _full((LATENT, H)),
              _full((1, H)),
              _full((H, 2 * H)),
              _full((1, 2 * H)),
              _full((2 * H, C_IN)),
              _full((1, C_IN)),
          ],
          out_specs=[
              pl.BlockSpec((B, LATENT), lambda p, n: (0, 0)),
              pl.BlockSpec((KN, C_IN), lambda p, n: (n, 0)),
          ],
          out_shape=[
              jax.ShapeDtypeStruct((B, LATENT), F32),
              jax.ShapeDtypeStruct((N, C_IN), F32),
          ],
          scratch_shapes=[pltpu.VMEM((B, H), F32)],
      )(h, agg2, batch_col, Wupd[t], bupd[t].reshape(1, H), W_out, b_out_r,
        Wd1, bd1_r, Wd2, bd2_r, Wd3, bd3_r)

  return (x_recon, z)


# speedup vs baseline: 5.2383x; 1.2903x over previous
import functools

import jax
import jax.numpy as jnp
import numpy as np
from jax import lax
from jax.experimental import pallas as pl
from jax.experimental.pallas import tpu as pltpu
from jax.experimental.pallas import tpu_sc as plsc

F32 = jnp.float32


def _make_posgather(N, E, KG=None):
  EPT = E // 32
  KG = KG or next(k for k in (1000, 500, 250, 125) if EPT % k == 0)
  NBLK = EPT // KG
  mesh = plsc.VectorSubcoreMesh(core_axis_name="c", subcore_axis_name="s", num_cores=2, num_subcores=16)

  @functools.partial(
      pl.kernel,
      out_type=[jax.ShapeDtypeStruct((E, 16), F32) for _ in range(4)],
      mesh=mesh,
      compiler_params=pltpu.CompilerParams(use_tc_tiling_on_sc=False),
      scratch_types=[
          [pltpu.VMEM((KG,), jnp.int32) for _ in range(4)],
          [pltpu.VMEM((KG, 16), F32) for _ in range(4)],
          pltpu.SemaphoreType.DMA,
          pltpu.SemaphoreType.DMA,
          pltpu.SemaphoreType.DMA,
      ],
  )
  def posgather(pos16_hbm, ih0, ih1, ih2, ih3, o0, o1, o2, o3, idx_v, bufs,
                sem_i, sem_g, sem_w):
    c = lax.axis_index("c")
    s = lax.axis_index("s")
    wid = s * 2 + c
    idxs = (ih0, ih1, ih2, ih3)
    outs = (o0, o1, o2, o3)

    def blk(b, carry):
      base = wid * EPT + b * KG
      ic = [pltpu.async_copy(idxs[g].at[pl.ds(base, KG)], idx_v[g], sem_i)
            for g in range(4)]
      for cp in ic:
        cp.wait()
      gc = [pltpu.async_copy(pos16_hbm.at[idx_v[g]], bufs[g], sem_g)
            for g in range(4)]
      for cp in gc:
        cp.wait()
      wc = [pltpu.async_copy(bufs[g], outs[g].at[pl.ds(base, KG)], sem_w)
            for g in range(4)]
      for cp in wc:
        cp.wait()
      return carry

    lax.fori_loop(0, NBLK, blk, 0)

  return posgather


def _make_msg(N, E, K=None):
  K = K or next(k for k in (400, 200, 100, 50) if (E // 16) % k == 0)
  NBLK_E = (E // 16) // K
  NPT = N // 16
  mesh = plsc.VectorSubcoreMesh(core_axis_name="c", subcore_axis_name="s", num_cores=2, num_subcores=16)

  @functools.partial(
      pl.kernel,
      out_type=jax.ShapeDtypeStruct((2, N, 32), F32),
      mesh=mesh,
      compiler_params=pltpu.CompilerParams(use_tc_tiling_on_sc=False),
      scratch_types=[
          pltpu.VMEM((K,), jnp.int32),
          pltpu.VMEM((K,), jnp.int32),
          pltpu.VMEM((K, 32), F32),
          pltpu.VMEM((K // 4, 128), F32),
          pltpu.VMEM((NPT // 25, 32), F32),
          pltpu.VMEM_SHARED((N, 32), F32),
          pltpu.SemaphoreType.DMA,
      ],
  )
  def msg(hw2, ghe2, jidx, iidx, out2, jv, iv, rows, ghb, zerob, agg_sp, sem):
    c = lax.axis_index("c")
    s = lax.axis_index("s")

    zch = NPT // 25

    def zset(r, carry):
      zero = jnp.zeros((16,), F32)
      for u in range(2):
        zerob[r, pl.ds(u * 16, 16)] = zero
      return carry

    lax.fori_loop(0, zch, zset, 0)
    for q in range(25):
      pltpu.sync_copy(zerob, agg_sp.at[pl.ds(s * NPT + q * zch, zch)])
    plsc.subcore_barrier()

    def edge_loop(hw_h, ghe_h):
      def blk(b, carry):
        base = s * (E // 16) + b * K
        cj = pltpu.async_copy(jidx.at[pl.ds(base, K)], jv, sem)
        ci = pltpu.async_copy(iidx.at[pl.ds(base, K)], iv, sem)
        cg = pltpu.async_copy(ghe_h.at[pl.ds(base // 4, K // 4)], ghb, sem)
        cj.wait()
        cr = pltpu.async_copy(hw_h.at[jv], rows, sem)
        ci.wait()
        cg.wait()
        cr.wait()

        def mul(r2, carry2):
          for dr in range(2):
            rr = r2 * 2 + dr
            for q in range(4):
              for u in range(2):
                sl = pl.ds(u * 16, 16)
                gl = pl.ds(q * 32 + u * 16, 16)
                rows[rr * 4 + q, sl] = rows[rr * 4 + q, sl] * ghb[rr, gl]
          return carry2

        lax.fori_loop(0, K // 8, mul, 0)
        pltpu.sync_copy(rows, agg_sp.at[iv], add=True)
        return carry

      lax.fori_loop(0, NBLK_E, blk, 0)

    for cc in (0, 1):
      @pl.when(c == cc)
      def _():
        edge_loop(hw2.at[cc], ghe2.at[cc])

    plsc.subcore_barrier()

    for cc in (0, 1):
      @pl.when(c == cc)
      def _():
        pltpu.sync_copy(agg_sp.at[pl.ds(s * NPT, NPT)],
                        out2.at[cc, pl.ds(s * NPT, NPT)])

  return msg


def _enc_in_body(x_ref, win_ref, bin_ref, wmsg_ref, h_ref, hw_ref):
  h = jnp.dot(x_ref[...], win_ref[...], preferred_element_type=F32) + bin_ref[...]
  h_ref[...] = h
  hw = jnp.dot(h, wmsg_ref[...], preferred_element_type=F32)
  hw_ref[...] = jnp.stack([hw[:, 0:32], hw[:, 32:64]], axis=0)


def _geo_body(pi_ref, pj_ref, pk_ref, pl_ref, w1_ref, b1_ref, w2_ref,
              b2_ref, senv_ref, o0, o1, o2, *, cutoff):
  pi = pi_ref[...].T
  pj = pj_ref[...].T
  pk = pk_ref[...].T
  plr = pl_ref[...].T
  v1 = pi - pj
  v2 = pj - pk
  v3 = pk - plr
  eps = 1e-8

  def dot3(a, b):
    return jnp.sum(a[0:3, :] * b[0:3, :], axis=0, keepdims=True)

  def cross(a, b):
    ax, ay, az = a[0:1, :], a[1:2, :], a[2:3, :]
    bx, by, bz = b[0:1, :], b[1:2, :], b[2:3, :]
    return jnp.concatenate(
        [ay * bz - az * by, az * bx - ax * bz, ax * by - ay * bx], axis=0)

  d1 = jnp.sqrt(dot3(v1, v1) + eps)
  d2 = jnp.sqrt(dot3(v2, v2) + eps)
  d3 = jnp.sqrt(dot3(v3, v3) + eps)
  cos_a = dot3(v1, v2) / (d1 * d2)
  cos_b = dot3(v2, v3) / (d2 * d3)
  n1 = cross(v1, v2)
  n2 = cross(v2, v3)
  n1n = jnp.sqrt(dot3(n1, n1) + eps)
  n2n = jnp.sqrt(dot3(n2, n2) + eps)
  cos_t = dot3(n1, n2) / (n1n * n2n)
  env = 0.5 * (jnp.cos(jnp.pi * jnp.minimum(d1, cutoff) / cutoff) + 1.0)
  env = env * (d1 < cutoff).astype(F32)
  zero = jnp.zeros_like(d1)
  g8 = jnp.concatenate([d1, d2, d3, cos_a, cos_b, cos_t, env, zero],
                       axis=0)
  kh = g8.shape[1] // 2
  geo_p = jnp.concatenate([g8[:, 0:kh], g8[:, kh:]], axis=0).T
  env_p = jnp.dot(geo_p, senv_ref[...], preferred_element_type=F32)
  for t, out in enumerate((o0, o1, o2)):
    g1 = jnp.maximum(
        jnp.dot(geo_p, w1_ref[t], preferred_element_type=F32)
        + b1_ref[t][None, :], 0.0)
    g2 = (jnp.dot(g1, w2_ref[t], preferred_element_type=F32)
          + b2_ref[t][None, :])
    ghe = g2 * env_p
    kq = kh // 2
    a0 = ghe[0:kq, :]
    a1 = ghe[kq:kh, :]
    out[...] = jnp.stack([
        jnp.concatenate([a0[:, 0:32], a1[:, 0:32],
                         a0[:, 64:96], a1[:, 64:96]], axis=1),
        jnp.concatenate([a0[:, 32:64], a1[:, 32:64],
                         a0[:, 96:128], a1[:, 96:128]], axis=1)], axis=0)


def _upd_body(h_ref, agg_ref, wupd_ref, bupd_ref, wmsg_ref, h_out, hw_out):
  agg = jnp.concatenate([agg_ref[0], agg_ref[1]], axis=1)
  upd = jnp.maximum(
      jnp.dot(agg, wupd_ref[...], preferred_element_type=F32) + bupd_ref[...],
      0.0)
  h = h_ref[...] + upd
  h_out[...] = h
  if hw_out is not None:
    hw = jnp.dot(h, wmsg_ref[...], preferred_element_type=F32)
    hw_out[...] = jnp.stack([hw[:, 0:32], hw[:, 32:64]], axis=0)


def _tail_body(h_ref, agg_ref, b_ref, wupd_ref, bupd_ref, wout_ref, bout_ref,
               wd1_ref, bd1_ref, wd2_ref, bd2_ref, wd3_ref, bd3_ref,
               z_ref, xr_ref, zg, *, nsteps, B):
  ph = pl.program_id(0)
  n = pl.program_id(1)
  onehot = (b_ref[...] == lax.broadcasted_iota(jnp.int32, (1, B), 1)
            ).astype(F32)

  @pl.when(ph == 0)
  def _():
    agg = jnp.concatenate([agg_ref[0], agg_ref[1]], axis=1)
    upd = jnp.maximum(
        jnp.dot(agg, wupd_ref[...], preferred_element_type=F32)
        + bupd_ref[...], 0.0)
    h = h_ref[...] + upd
    contrib = lax.dot_general(onehot, h, (((0,), (0,)), ((), ())),
                              preferred_element_type=F32)

    @pl.when(n == 0)
    def _():
      zg[...] = jnp.zeros_like(zg)

    zg[...] += contrib

    @pl.when(n == nsteps - 1)
    def _():
      z_ref[...] = (jnp.dot(zg[...], wout_ref[...],
                            preferred_element_type=F32) + bout_ref[...])

  @pl.when(ph == 1)
  def _():
    z = (jnp.dot(zg[...], wout_ref[...], preferred_element_type=F32)
         + bout_ref[...])
    zx = jnp.dot(onehot, z, preferred_element_type=F32)
    hd = jnp.maximum(
        jnp.dot(zx, wd1_ref[...], preferred_element_type=F32)
        + bd1_ref[...], 0.0)
    hd = jnp.maximum(
        jnp.dot(hd, wd2_ref[...], preferred_element_type=F32)
        + bd2_ref[...], 0.0)
    xr_ref[...] = (jnp.dot(hd, wd3_ref[...], preferred_element_type=F32)
                   + bd3_ref[...])


def _readout_body(h_ref, b_ref, wout_ref, bout_ref, z_ref, zg, *, nsteps, B):
  step = pl.program_id(0)
  onehot = (b_ref[...] == lax.broadcasted_iota(jnp.int32, (1, B), 1)
            ).astype(F32)
  contrib = lax.dot_general(onehot, h_ref[...], (((0,), (0,)), ((), ())),
                            preferred_element_type=F32)

  @pl.when(step == 0)
  def _():
    zg[...] = jnp.zeros_like(zg)

  zg[...] += contrib

  @pl.when(step == nsteps - 1)
  def _():
    z_ref[...] = (jnp.dot(zg[...], wout_ref[...], preferred_element_type=F32)
                  + bout_ref[...])


def _dec_body(z_ref, b_ref, wd1_ref, bd1_ref, wd2_ref, bd2_ref, wd3_ref,
              bd3_ref, out_ref, *, B):
  onehot = (b_ref[...] == lax.broadcasted_iota(jnp.int32, (1, B), 1)
            ).astype(F32)
  zx = jnp.dot(onehot, z_ref[...], preferred_element_type=F32)
  hd = jnp.maximum(
      jnp.dot(zx, wd1_ref[...], preferred_element_type=F32) + bd1_ref[...], 0.0)
  hd = jnp.maximum(
      jnp.dot(hd, wd2_ref[...], preferred_element_type=F32) + bd2_ref[...], 0.0)
  out_ref[...] = (jnp.dot(hd, wd3_ref[...], preferred_element_type=F32)
                  + bd3_ref[...])


def _full(shape, dtype=F32):
  return pl.BlockSpec(shape, lambda *_: tuple(0 for _ in shape))


def kernel(x, pos, batch, edge_index_3rd, num_nodes_per_graph,
           W_in, b_in, Wg1, bg1, Wg2, bg2, Wmsg, Wupd, bupd,
           W_out, b_out, Wd1, bd1, Wd2, bd2, Wd3, bd3):
  N, C_IN = x.shape
  E = edge_index_3rd.shape[1]
  H = W_in.shape[1]
  LATENT = W_out.shape[1]
  T = Wmsg.shape[0]
  B = num_nodes_per_graph.shape[0]
  CUTOFF = 10.0

  KN = next(k for k in (2000, 1600, 800, 400, 200, 100) if N % k == 0)
  NSTEPS = N // KN
  KT = next(k for k in (3200, 1600, 800, 400) if E % k == 0)
  ESTEPS = E // KT

  pos16 = jnp.pad(pos, ((0, 0), (0, 13)))
  batch_col = batch.reshape(N, 1)
  jidx = edge_index_3rd[1]
  iidx = edge_index_3rd[0]
  def blockdiag2(w):
    z = jnp.zeros_like(w)
    return jnp.concatenate([
        jnp.concatenate([w, z], axis=1),
        jnp.concatenate([z, w], axis=1)], axis=0)

  Wg1p = jnp.pad(Wg1, ((0, 0), (0, 2), (0, 0)))
  W1big = jnp.stack([blockdiag2(Wg1p[t]) for t in range(T)])
  W2big = jnp.stack([blockdiag2(Wg2[t]) for t in range(T)])
  b1big = jnp.concatenate([bg1, bg1], axis=1)
  b2big = jnp.concatenate([bg2, bg2], axis=1)
  Senv = jnp.zeros((16, 2 * H), F32)
  Senv = Senv.at[6, 0:H].set(1.0).at[14, H:2 * H].set(1.0)
  b_in_r = b_in.reshape(1, H)
  b_out_r = b_out.reshape(1, LATENT)
  bd1_r = bd1.reshape(1, -1)
  bd2_r = bd2.reshape(1, -1)
  bd3_r = bd3.reshape(1, -1)

  h0, hw0 = pl.pallas_call(
      _enc_in_body,
      grid=(NSTEPS,),
      in_specs=[
          pl.BlockSpec((KN, C_IN), lambda n: (n, 0)),
          _full((C_IN, H)),
          _full((1, H)),
          _full((H, H)),
      ],
      out_specs=[
          pl.BlockSpec((KN, H), lambda n: (n, 0)),
          pl.BlockSpec((2, KN, 32), lambda n: (0, n, 0)),
      ],
      out_shape=[
          jax.ShapeDtypeStruct((N, H), F32),
          jax.ShapeDtypeStruct((2, N, 32), F32),
      ],
  )(x, W_in, b_in_r, Wmsg[0])

  kidx = edge_index_3rd[2]
  lidx = edge_index_3rd[3]
  pgi, pgj, pgk, pgl = _make_posgather(N, E)(pos16, iidx, jidx, kidx, lidx)

  ghe_all = pl.pallas_call(
      functools.partial(_geo_body, cutoff=CUTOFF),
      grid=(ESTEPS,),
      in_specs=[
          pl.BlockSpec((KT, 16), lambda e: (e, 0)),
          pl.BlockSpec((KT, 16), lambda e: (e, 0)),
          pl.BlockSpec((KT, 16), lambda e: (e, 0)),
          pl.BlockSpec((KT, 16), lambda e: (e, 0)),
          _full((T, 16, 2 * H)),
          _full((T, 2 * H)),
          _full((T, 2 * H, 2 * H)),
          _full((T, 2 * H)),
          _full((16, 2 * H)),
      ],
      out_specs=[pl.BlockSpec((2, KT // 4, 128), lambda e: (0, e, 0))] * 3,
      out_shape=[jax.ShapeDtypeStruct((2, E // 4, 128), F32)] * 3,
  )(pgi, pgj, pgk, pgl, W1big, b1big, W2big, b2big, Senv)

  p = np.arange(E)
  bblk = p // KT
  pp = p % KT
  perm = jnp.asarray(bblk * KT + (pp // 4) + (KT // 4) * (pp % 4),
                     dtype=jnp.int32)
  jidx = jnp.take(jidx, perm)
  iidx_p = jnp.take(iidx, perm)

  msg_call = _make_msg(N, E)
  h = h0
  hw2 = hw0
  for t in range(T):
    agg2 = msg_call(hw2, ghe_all[t], jidx, iidx_p)
    last = (t == T - 1)
    if not last:
      h, hw2 = pl.pallas_call(
          _upd_body,
          grid=(NSTEPS,),
          in_specs=[
              pl.BlockSpec((KN, H), lambda n: (n, 0)),
              pl.BlockSpec((2, KN, 32), lambda n: (0, n, 0)),
              _full((H, H)),
              _full((1, H)),
              _full((H, H)),
          ],
          out_specs=[
              pl.BlockSpec((KN, H), lambda n: (n, 0)),
              pl.BlockSpec((2, KN, 32), lambda n: (0, n, 0)),
          ],
          out_shape=[
              jax.ShapeDtypeStruct((N, H), F32),
              jax.ShapeDtypeStruct((2, N, 32), F32),
          ],
      )(h, agg2, Wupd[t], bupd[t].reshape(1, H), Wmsg[t + 1])
    else:
      z, x_recon = pl.pallas_call(
          functools.partial(_tail_body, nsteps=NSTEPS, B=B),
          grid=(2, NSTEPS),
          in_specs=[
              pl.BlockSpec((KN, H), lambda p, n: (n, 0)),
              pl.BlockSpec((2, KN, 32), lambda p, n: (0, n, 0)),
              pl.BlockSpec((KN, 1), lambda p, n: (n, 0)),
              _full((H, H)),
              _full((1, H)),
              _full((H, LATENT)),
              _full((1, LATENT)),
              _full((LATENT, H)),
              _full((1, H)),
              _full((H, 2 * H)),
              _full((1, 2 * H)),
              _full((2 * H, C_IN)),
              _full((1, C_IN)),
          ],
          out_specs=[
              pl.BlockSpec((B, LATENT), lambda p, n: (0, 0)),
              pl.BlockSpec((KN, C_IN), lambda p, n: (n, 0)),
          ],
          out_shape=[
              jax.ShapeDtypeStruct((B, LATENT), F32),
              jax.ShapeDtypeStruct((N, C_IN), F32),
          ],
          scratch_shapes=[pltpu.VMEM((B, H), F32)],
      )(h, agg2, batch_col, Wupd[t], bupd[t].reshape(1, H), W_out, b_out_r,
        Wd1, bd1_r, Wd2, bd2_r, Wd3, bd3_r)

  return (x_recon, z)


# speedup vs baseline: 7.5633x; 1.4438x over previous
import functools

import jax
import jax.numpy as jnp
import numpy as np
from jax import lax
from jax.experimental import pallas as pl
from jax.experimental.pallas import tpu as pltpu
from jax.experimental.pallas import tpu_sc as plsc

F32 = jnp.float32


def _make_posgather(N, E, KG=None):
  EPT = E // 32
  KG = KG or next(k for k in (1000, 500, 250, 125) if EPT % k == 0)
  NBLK = EPT // KG
  mesh = plsc.VectorSubcoreMesh(core_axis_name="c", subcore_axis_name="s", num_cores=2, num_subcores=16)

  @functools.partial(
      pl.kernel,
      out_type=[jax.ShapeDtypeStruct((E, 16), F32) for _ in range(4)],
      mesh=mesh,
      compiler_params=pltpu.CompilerParams(use_tc_tiling_on_sc=False),
      scratch_types=[
          [pltpu.VMEM((KG,), jnp.int32) for _ in range(4)],
          [pltpu.VMEM((KG, 16), F32) for _ in range(4)],
          pltpu.SemaphoreType.DMA,
          pltpu.SemaphoreType.DMA,
          pltpu.SemaphoreType.DMA,
      ],
  )
  def posgather(pos16_hbm, ih0, ih1, ih2, ih3, o0, o1, o2, o3, idx_v, bufs,
                sem_i, sem_g, sem_w):
    c = lax.axis_index("c")
    s = lax.axis_index("s")
    wid = s * 2 + c
    idxs = (ih0, ih1, ih2, ih3)
    outs = (o0, o1, o2, o3)

    def blk(b, carry):
      base = wid * EPT + b * KG
      ic = [pltpu.async_copy(idxs[g].at[pl.ds(base, KG)], idx_v[g], sem_i)
            for g in range(4)]
      for cp in ic:
        cp.wait()
      gc = [pltpu.async_copy(pos16_hbm.at[idx_v[g]], bufs[g], sem_g)
            for g in range(4)]
      for cp in gc:
        cp.wait()
      wc = [pltpu.async_copy(bufs[g], outs[g].at[pl.ds(base, KG)], sem_w)
            for g in range(4)]
      for cp in wc:
        cp.wait()
      return carry

    lax.fori_loop(0, NBLK, blk, 0)

  return posgather


def _make_msg(N, E, K=None):
  K = K or next(k for k in (400, 200, 100, 50) if (E // 16) % k == 0)
  NBLK_E = (E // 16) // K
  NPT = N // 16
  mesh = plsc.VectorSubcoreMesh(core_axis_name="c", subcore_axis_name="s", num_cores=2, num_subcores=16)

  @functools.partial(
      pl.kernel,
      out_type=jax.ShapeDtypeStruct((2, N, 32), F32),
      mesh=mesh,
      compiler_params=pltpu.CompilerParams(use_tc_tiling_on_sc=False),
      scratch_types=[
          pltpu.VMEM((K,), jnp.int32),
          pltpu.VMEM((K,), jnp.int32),
          pltpu.VMEM((K, 32), F32),
          pltpu.VMEM((K // 4, 128), F32),
          pltpu.VMEM((NPT // 25, 32), F32),
          pltpu.VMEM_SHARED((N, 32), F32),
          pltpu.SemaphoreType.DMA,
      ],
  )
  def msg(hw2, ghe2, jidx, iidx, out2, jv, iv, rows, ghb, zerob, agg_sp, sem):
    c = lax.axis_index("c")
    s = lax.axis_index("s")

    zch = NPT // 25

    def zset(r, carry):
      zero = jnp.zeros((16,), F32)
      for u in range(2):
        zerob[r, pl.ds(u * 16, 16)] = zero
      return carry

    lax.fori_loop(0, zch, zset, 0)
    for q in range(25):
      pltpu.sync_copy(zerob, agg_sp.at[pl.ds(s * NPT + q * zch, zch)])
    plsc.subcore_barrier()

    def edge_loop(hw_h, ghe_h):
      def blk(b, carry):
        base = s * (E // 16) + b * K
        cj = pltpu.async_copy(jidx.at[pl.ds(base, K)], jv, sem)
        ci = pltpu.async_copy(iidx.at[pl.ds(base, K)], iv, sem)
        cg = pltpu.async_copy(ghe_h.at[pl.ds(base // 4, K // 4)], ghb, sem)
        cj.wait()
        cr = pltpu.async_copy(hw_h.at[jv], rows, sem)
        ci.wait()
        cg.wait()
        cr.wait()

        def mul(r2, carry2):
          for dr in range(2):
            rr = r2 * 2 + dr
            for q in range(4):
              for u in range(2):
                sl = pl.ds(u * 16, 16)
                gl = pl.ds(q * 32 + u * 16, 16)
                rows[rr * 4 + q, sl] = rows[rr * 4 + q, sl] * ghb[rr, gl]
          return carry2

        lax.fori_loop(0, K // 8, mul, 0)
        pltpu.sync_copy(rows, agg_sp.at[iv], add=True)
        return carry

      lax.fori_loop(0, NBLK_E, blk, 0)

    for cc in (0, 1):
      @pl.when(c == cc)
      def _():
        edge_loop(hw2.at[cc], ghe2.at[cc])

    plsc.subcore_barrier()

    for cc in (0, 1):
      @pl.when(c == cc)
      def _():
        pltpu.sync_copy(agg_sp.at[pl.ds(s * NPT, NPT)],
                        out2.at[cc, pl.ds(s * NPT, NPT)])

  return msg


def _enc_in_body(x_ref, win_ref, bin_ref, wmsg_ref, h_ref, hw_ref):
  h = jnp.dot(x_ref[...], win_ref[...], preferred_element_type=F32) + bin_ref[...]
  h_ref[...] = h
  hw = jnp.dot(h, wmsg_ref[...], preferred_element_type=F32)
  hw_ref[...] = jnp.stack([hw[:, 0:32], hw[:, 32:64]], axis=0)


def _geo_body(pi_ref, pj_ref, pk_ref, pl_ref, w1_ref, b1_ref, w2_ref,
              b2_ref, senv_ref, o0, o1, o2, *, cutoff):
  def unpack(ref):
    xt = ref[...].T
    return jnp.concatenate([xt[16 * k:16 * (k + 1), :] for k in range(8)],
                           axis=1)

  pi = unpack(pi_ref)
  pj = unpack(pj_ref)
  pk = unpack(pk_ref)
  plr = unpack(pl_ref)
  v1 = pi - pj
  v2 = pj - pk
  v3 = pk - plr
  eps = 1e-8

  def dot3(a, b):
    return jnp.sum(a[0:3, :] * b[0:3, :], axis=0, keepdims=True)

  def cross(a, b):
    ax, ay, az = a[0:1, :], a[1:2, :], a[2:3, :]
    bx, by, bz = b[0:1, :], b[1:2, :], b[2:3, :]
    return jnp.concatenate(
        [ay * bz - az * by, az * bx - ax * bz, ax * by - ay * bx], axis=0)

  d1 = jnp.sqrt(dot3(v1, v1) + eps)
  d2 = jnp.sqrt(dot3(v2, v2) + eps)
  d3 = jnp.sqrt(dot3(v3, v3) + eps)
  cos_a = dot3(v1, v2) / (d1 * d2)
  cos_b = dot3(v2, v3) / (d2 * d3)
  n1 = cross(v1, v2)
  n2 = cross(v2, v3)
  n1n = jnp.sqrt(dot3(n1, n1) + eps)
  n2n = jnp.sqrt(dot3(n2, n2) + eps)
  cos_t = dot3(n1, n2) / (n1n * n2n)
  env = 0.5 * (jnp.cos(jnp.pi * jnp.minimum(d1, cutoff) / cutoff) + 1.0)
  env = env * (d1 < cutoff).astype(F32)
  zero = jnp.zeros_like(d1)
  g8 = jnp.concatenate([d1, d2, d3, cos_a, cos_b, cos_t, env, zero],
                       axis=0)
  kh = g8.shape[1] // 2
  geo_p = jnp.concatenate([g8[:, 0:kh], g8[:, kh:]], axis=0).T
  env_p = jnp.dot(geo_p, senv_ref[...], preferred_element_type=F32)
  for t, out in enumerate((o0, o1, o2)):
    g1 = jnp.maximum(
        jnp.dot(geo_p, w1_ref[t], preferred_element_type=F32)
        + b1_ref[t][None, :], 0.0)
    g2 = (jnp.dot(g1, w2_ref[t], preferred_element_type=F32)
          + b2_ref[t][None, :])
    ghe = g2 * env_p
    kq = kh // 2
    a0 = ghe[0:kq, :]
    a1 = ghe[kq:kh, :]
    out[...] = jnp.stack([
        jnp.concatenate([a0[:, 0:32], a1[:, 0:32],
                         a0[:, 64:96], a1[:, 64:96]], axis=1),
        jnp.concatenate([a0[:, 32:64], a1[:, 32:64],
                         a0[:, 96:128], a1[:, 96:128]], axis=1)], axis=0)


def _upd_body(h_ref, agg_ref, wupd_ref, bupd_ref, wmsg_ref, h_out, hw_out):
  agg = jnp.concatenate([agg_ref[0], agg_ref[1]], axis=1)
  upd = jnp.maximum(
      jnp.dot(agg, wupd_ref[...], preferred_element_type=F32) + bupd_ref[...],
      0.0)
  h = h_ref[...] + upd
  h_out[...] = h
  if hw_out is not None:
    hw = jnp.dot(h, wmsg_ref[...], preferred_element_type=F32)
    hw_out[...] = jnp.stack([hw[:, 0:32], hw[:, 32:64]], axis=0)


def _tail_body(h_ref, agg_ref, b_ref, wupd_ref, bupd_ref, wout_ref, bout_ref,
               wd1_ref, bd1_ref, wd2_ref, bd2_ref, wd3_ref, bd3_ref,
               z_ref, xr_ref, zg, *, nsteps, B):
  ph = pl.program_id(0)
  n = pl.program_id(1)
  onehot = (b_ref[...] == lax.broadcasted_iota(jnp.int32, (1, B), 1)
            ).astype(F32)

  @pl.when(ph == 0)
  def _():
    agg = jnp.concatenate([agg_ref[0], agg_ref[1]], axis=1)
    upd = jnp.maximum(
        jnp.dot(agg, wupd_ref[...], preferred_element_type=F32)
        + bupd_ref[...], 0.0)
    h = h_ref[...] + upd
    contrib = lax.dot_general(onehot, h, (((0,), (0,)), ((), ())),
                              preferred_element_type=F32)

    @pl.when(n == 0)
    def _():
      zg[...] = jnp.zeros_like(zg)

    zg[...] += contrib

    @pl.when(n == nsteps - 1)
    def _():
      z_ref[...] = (jnp.dot(zg[...], wout_ref[...],
                            preferred_element_type=F32) + bout_ref[...])

  @pl.when(ph == 1)
  def _():
    z = (jnp.dot(zg[...], wout_ref[...], preferred_element_type=F32)
         + bout_ref[...])
    zx = jnp.dot(onehot, z, preferred_element_type=F32)
    hd = jnp.maximum(
        jnp.dot(zx, wd1_ref[...], preferred_element_type=F32)
        + bd1_ref[...], 0.0)
    hd = jnp.maximum(
        jnp.dot(hd, wd2_ref[...], preferred_element_type=F32)
        + bd2_ref[...], 0.0)
    xr_ref[...] = (jnp.dot(hd, wd3_ref[...], preferred_element_type=F32)
                   + bd3_ref[...])


def _readout_body(h_ref, b_ref, wout_ref, bout_ref, z_ref, zg, *, nsteps, B):
  step = pl.program_id(0)
  onehot = (b_ref[...] == lax.broadcasted_iota(jnp.int32, (1, B), 1)
            ).astype(F32)
  contrib = lax.dot_general(onehot, h_ref[...], (((0,), (0,)), ((), ())),
                            preferred_element_type=F32)

  @pl.when(step == 0)
  def _():
    zg[...] = jnp.zeros_like(zg)

  zg[...] += contrib

  @pl.when(step == nsteps - 1)
  def _():
    z_ref[...] = (jnp.dot(zg[...], wout_ref[...], preferred_element_type=F32)
                  + bout_ref[...])


def _dec_body(z_ref, b_ref, wd1_ref, bd1_ref, wd2_ref, bd2_ref, wd3_ref,
              bd3_ref, out_ref, *, B):
  onehot = (b_ref[...] == lax.broadcasted_iota(jnp.int32, (1, B), 1)
            ).astype(F32)
  zx = jnp.dot(onehot, z_ref[...], preferred_element_type=F32)
  hd = jnp.maximum(
      jnp.dot(zx, wd1_ref[...], preferred_element_type=F32) + bd1_ref[...], 0.0)
  hd = jnp.maximum(
      jnp.dot(hd, wd2_ref[...], preferred_element_type=F32) + bd2_ref[...], 0.0)
  out_ref[...] = (jnp.dot(hd, wd3_ref[...], preferred_element_type=F32)
                  + bd3_ref[...])


def _full(shape, dtype=F32):
  return pl.BlockSpec(shape, lambda *_: tuple(0 for _ in shape))


def kernel(x, pos, batch, edge_index_3rd, num_nodes_per_graph,
           W_in, b_in, Wg1, bg1, Wg2, bg2, Wmsg, Wupd, bupd,
           W_out, b_out, Wd1, bd1, Wd2, bd2, Wd3, bd3):
  N, C_IN = x.shape
  E = edge_index_3rd.shape[1]
  H = W_in.shape[1]
  LATENT = W_out.shape[1]
  T = Wmsg.shape[0]
  B = num_nodes_per_graph.shape[0]
  CUTOFF = 10.0

  KN = next(k for k in (2000, 1600, 800, 400, 200, 100) if N % k == 0)
  NSTEPS = N // KN
  KT = next(k for k in (3200, 1600, 800, 400) if E % k == 0)
  ESTEPS = E // KT

  pos16 = jnp.pad(pos, ((0, 0), (0, 13)))
  batch_col = batch.reshape(N, 1)
  jidx = edge_index_3rd[1]
  iidx = edge_index_3rd[0]
  def blockdiag2(w):
    z = jnp.zeros_like(w)
    return jnp.concatenate([
        jnp.concatenate([w, z], axis=1),
        jnp.concatenate([z, w], axis=1)], axis=0)

  Wg1p = jnp.pad(Wg1, ((0, 0), (0, 2), (0, 0)))
  W1big = jnp.stack([blockdiag2(Wg1p[t]) for t in range(T)])
  W2big = jnp.stack([blockdiag2(Wg2[t]) for t in range(T)])
  b1big = jnp.concatenate([bg1, bg1], axis=1)
  b2big = jnp.concatenate([bg2, bg2], axis=1)
  Senv = jnp.zeros((16, 2 * H), F32)
  Senv = Senv.at[6, 0:H].set(1.0).at[14, H:2 * H].set(1.0)
  b_in_r = b_in.reshape(1, H)
  b_out_r = b_out.reshape(1, LATENT)
  bd1_r = bd1.reshape(1, -1)
  bd2_r = bd2.reshape(1, -1)
  bd3_r = bd3.reshape(1, -1)

  h0, hw0 = pl.pallas_call(
      _enc_in_body,
      grid=(NSTEPS,),
      in_specs=[
          pl.BlockSpec((KN, C_IN), lambda n: (n, 0)),
          _full((C_IN, H)),
          _full((1, H)),
          _full((H, H)),
      ],
      out_specs=[
          pl.BlockSpec((KN, H), lambda n: (n, 0)),
          pl.BlockSpec((2, KN, 32), lambda n: (0, n, 0)),
      ],
      out_shape=[
          jax.ShapeDtypeStruct((N, H), F32),
          jax.ShapeDtypeStruct((2, N, 32), F32),
      ],
  )(x, W_in, b_in_r, Wmsg[0])

  kidx = edge_index_3rd[2]
  lidx = edge_index_3rd[3]
  pgi, pgj, pgk, pgl = _make_posgather(N, E)(pos16, iidx, jidx, kidx, lidx)
  pgi, pgj, pgk, pgl = (a.reshape(E // 8, 128) for a in (pgi, pgj, pgk, pgl))

  ghe_all = pl.pallas_call(
      functools.partial(_geo_body, cutoff=CUTOFF),
      grid=(ESTEPS,),
      in_specs=[
          pl.BlockSpec((KT // 8, 128), lambda e: (e, 0)),
          pl.BlockSpec((KT // 8, 128), lambda e: (e, 0)),
          pl.BlockSpec((KT // 8, 128), lambda e: (e, 0)),
          pl.BlockSpec((KT // 8, 128), lambda e: (e, 0)),
          _full((T, 16, 2 * H)),
          _full((T, 2 * H)),
          _full((T, 2 * H, 2 * H)),
          _full((T, 2 * H)),
          _full((16, 2 * H)),
      ],
      out_specs=[pl.BlockSpec((2, KT // 4, 128), lambda e: (0, e, 0))] * 3,
      out_shape=[jax.ShapeDtypeStruct((2, E // 4, 128), F32)] * 3,
  )(pgi, pgj, pgk, pgl, W1big, b1big, W2big, b2big, Senv)

  p = np.arange(E)
  bblk = p // KT
  pp = p % KT
  m = (pp // 4) + (KT // 4) * (pp % 4)
  perm = jnp.asarray(bblk * KT + 8 * (m % (KT // 8)) + (m // (KT // 8)),
                     dtype=jnp.int32)
  jidx = jnp.take(jidx, perm)
  iidx_p = jnp.take(iidx, perm)

  msg_call = _make_msg(N, E)
  h = h0
  hw2 = hw0
  for t in range(T):
    agg2 = msg_call(hw2, ghe_all[t], jidx, iidx_p)
    last = (t == T - 1)
    if not last:
      h, hw2 = pl.pallas_call(
          _upd_body,
          grid=(NSTEPS,),
          in_specs=[
              pl.BlockSpec((KN, H), lambda n: (n, 0)),
              pl.BlockSpec((2, KN, 32), lambda n: (0, n, 0)),
              _full((H, H)),
              _full((1, H)),
              _full((H, H)),
          ],
          out_specs=[
              pl.BlockSpec((KN, H), lambda n: (n, 0)),
              pl.BlockSpec((2, KN, 32), lambda n: (0, n, 0)),
          ],
          out_shape=[
              jax.ShapeDtypeStruct((N, H), F32),
              jax.ShapeDtypeStruct((2, N, 32), F32),
          ],
      )(h, agg2, Wupd[t], bupd[t].reshape(1, H), Wmsg[t + 1])
    else:
      z, x_recon = pl.pallas_call(
          functools.partial(_tail_body, nsteps=NSTEPS, B=B),
          grid=(2, NSTEPS),
          in_specs=[
              pl.BlockSpec((KN, H), lambda p, n: (n, 0)),
              pl.BlockSpec((2, KN, 32), lambda p, n: (0, n, 0)),
              pl.BlockSpec((KN, 1), lambda p, n: (n, 0)),
              _full((H, H)),
              _full((1, H)),
              _full((H, LATENT)),
              _full((1, LATENT)),
              _full((LATENT, H)),
              _full((1, H)),
              _full((H, 2 * H)),
              _full((1, 2 * H)),
              _full((2 * H, C_IN)),
              _full((1, C_IN)),
          ],
          out_specs=[
              pl.BlockSpec((B, LATENT), lambda p, n: (0, 0)),
              pl.BlockSpec((KN, C_IN), lambda p, n: (n, 0)),
          ],
          out_shape=[
              jax.ShapeDtypeStruct((B, LATENT), F32),
              jax.ShapeDtypeStruct((N, C_IN), F32),
          ],
          scratch_shapes=[pltpu.VMEM((B, H), F32)],
      )(h, agg2, batch_col, Wupd[t], bupd[t].reshape(1, H), W_out, b_out_r,
        Wd1, bd1_r, Wd2, bd2_r, Wd3, bd3_r)

  return (x_recon, z)
